# Initial kernel scaffold; baseline (speedup 1.0000x reference)
import jax
import jax.numpy as jnp
from jax.experimental import pallas as pl


def kernel(x, W_router, Wg, bg, Wv, bv, Wo, bo):
    raise NotImplementedError("write your pallas kernel here")


# speedup vs baseline: 2.0010x; 2.0010x over previous
import functools

import jax
import jax.numpy as jnp
from jax.experimental import pallas as pl
from jax.experimental.pallas import tpu as pltpu

MODEL_DIM = 768
FFN_DIM = 768
NUM_EXPERTS = 64
CAPACITY = 64
TOKENS = 2 * 2048
TB = 256


def _router_dispatch_body(x_ref, wr_ref, logits_ref, probs_ref, occ_ref,
                          tok_ref, ptb_ref, carry_ref):
    g = pl.program_id(0)
    E = NUM_EXPERTS

    @pl.when(g == 0)
    def _init():
        carry_ref[...] = jnp.zeros_like(carry_ref)
        occ_ref[...] = jnp.zeros_like(occ_ref)
        tok_ref[...] = jnp.zeros_like(tok_ref)
        ptb_ref[...] = jnp.zeros_like(ptb_ref)

    xb = x_ref[...]
    logits = jnp.dot(xb, wr_ref[...], preferred_element_type=jnp.float32)
    logits_ref[...] = logits
    m = jnp.max(logits, axis=1, keepdims=True)
    ex = jnp.exp(logits - m)
    probs = ex / jnp.sum(ex, axis=1, keepdims=True)
    probs_ref[...] = probs

    lane = jax.lax.broadcasted_iota(jnp.int32, (TB, E), 1)
    top_i = jnp.min(jnp.where(logits == m, lane, E), axis=1)
    top_p = jnp.max(probs, axis=1)
    oh_e = (lane == top_i[:, None]).astype(jnp.float32)

    cs = oh_e
    k = 1
    while k < TB:
        cs = cs + jnp.concatenate(
            [jnp.zeros((k, E), jnp.float32), cs[:-k, :]], axis=0)
        k *= 2
    pos_mat = cs - oh_e + carry_ref[0:1, :]
    carry_ref[0:1, :] = carry_ref[0:1, :] + cs[TB - 1:TB, :]
    pos_t = jnp.sum(pos_mat * oh_e, axis=1)

    cap_lane = jax.lax.broadcasted_iota(jnp.int32, (TB, CAPACITY), 1)
    pos_i = pos_t.astype(jnp.int32)
    oh_c = (cap_lane == pos_i[:, None]).astype(jnp.float32)
    tok_id = (jax.lax.broadcasted_iota(jnp.int32, (TB, 1), 0)
              + g * TB).astype(jnp.float32)
    dn = (((0,), (0,)), ((), ()))
    hi = jax.lax.Precision.HIGHEST
    occ_ref[...] += jax.lax.dot_general(
        oh_e, oh_c, dn, precision=hi, preferred_element_type=jnp.float32)
    tok_ref[...] += jax.lax.dot_general(
        oh_e, oh_c * tok_id, dn, precision=hi, preferred_element_type=jnp.float32)
    ptb_ref[...] += jax.lax.dot_general(
        oh_e, oh_c * top_p[:, None], dn, precision=hi,
        preferred_element_type=jnp.float32)


def _expert_ffn_body(idx_s, xf_ref, wg_ref, bg_ref, wv_ref, bv_ref,
                     wo_ref, bo_ref, w_ref, out_ref, xi_ref):
    e = pl.program_id(0)

    @pl.when(e == 0)
    def _init():
        out_ref[...] = jnp.zeros_like(out_ref)

    for i in range(CAPACITY):
        t = idx_s[e * CAPACITY + i]
        ts = jnp.maximum(t, 0)
        xi_ref[pl.ds(i, 1), :] = xf_ref[pl.ds(ts, 1), :]

    xi = xi_ref[...]
    gate = jnp.dot(xi, wg_ref[0], preferred_element_type=jnp.float32) + bg_ref[0, 0][None, :]
    val = jnp.dot(xi, wv_ref[0], preferred_element_type=jnp.float32) + bv_ref[0, 0][None, :]
    h = val * (gate * jax.nn.sigmoid(gate))
    eo = jnp.dot(h, wo_ref[0], preferred_element_type=jnp.float32) + bo_ref[0, 0][None, :]
    eo = eo * w_ref[0]

    for i in range(CAPACITY):
        t = idx_s[e * CAPACITY + i]
        row = eo[i:i + 1, :]

        @pl.when(t >= 0)
        def _store(row=row, t=t):
            out_ref[pl.ds(t, 1), :] = row


@jax.jit
def kernel(x, W_router, Wg, bg, Wv, bv, Wo, bo):
    B, S, D = x.shape
    T = B * S
    E, C, F = NUM_EXPERTS, CAPACITY, FFN_DIM
    xf = x.reshape(T, D)

    nblk = T // TB
    logits, probs, occ, tok, ptb = pl.pallas_call(
        _router_dispatch_body,
        grid=(nblk,),
        in_specs=[
            pl.BlockSpec((TB, D), lambda g: (g, 0)),
            pl.BlockSpec((D, E), lambda g: (0, 0)),
        ],
        out_specs=[
            pl.BlockSpec((TB, E), lambda g: (g, 0)),
            pl.BlockSpec((TB, E), lambda g: (g, 0)),
            pl.BlockSpec((E, C), lambda g: (0, 0)),
            pl.BlockSpec((E, C), lambda g: (0, 0)),
            pl.BlockSpec((E, C), lambda g: (0, 0)),
        ],
        out_shape=[
            jax.ShapeDtypeStruct((T, E), jnp.float32),
            jax.ShapeDtypeStruct((T, E), jnp.float32),
            jax.ShapeDtypeStruct((E, C), jnp.float32),
            jax.ShapeDtypeStruct((E, C), jnp.float32),
            jax.ShapeDtypeStruct((E, C), jnp.float32),
        ],
        scratch_shapes=[pltpu.VMEM((8, E), jnp.float32)],
    )(xf, W_router)

    expert_indices = jnp.where(occ > 0.5, tok.astype(jnp.int32), -1)
    expert_probs = ptb

    grid_spec = pltpu.PrefetchScalarGridSpec(
        num_scalar_prefetch=1,
        grid=(E,),
        in_specs=[
            pl.BlockSpec((T, D), lambda e, s: (0, 0)),
            pl.BlockSpec((1, D, F), lambda e, s: (e, 0, 0)),
            pl.BlockSpec((1, 1, F), lambda e, s: (e, 0, 0)),
            pl.BlockSpec((1, D, F), lambda e, s: (e, 0, 0)),
            pl.BlockSpec((1, 1, F), lambda e, s: (e, 0, 0)),
            pl.BlockSpec((1, F, D), lambda e, s: (e, 0, 0)),
            pl.BlockSpec((1, 1, D), lambda e, s: (e, 0, 0)),
            pl.BlockSpec((1, C, 1), lambda e, s: (e, 0, 0)),
        ],
        out_specs=pl.BlockSpec((T, D), lambda e, s: (0, 0)),
        scratch_shapes=[pltpu.VMEM((C, D), jnp.float32)],
    )
    out = pl.pallas_call(
        _expert_ffn_body,
        grid_spec=grid_spec,
        out_shape=jax.ShapeDtypeStruct((T, D), jnp.float32),
    )(expert_indices.reshape(-1), xf, Wg, bg.reshape(E, 1, F), Wv,
      bv.reshape(E, 1, F), Wo, bo.reshape(E, 1, D),
      expert_probs.reshape(E, C, 1))

    return (out.reshape(B, S, D), logits, probs, expert_probs, expert_indices)
